# baseline (speedup 1.0000x reference)
import jax
import jax.numpy as jnp
from jax import lax
from jax.experimental import pallas as pl
from jax.experimental.pallas import tpu as pltpu
from jax.experimental.pallas import tpu_sc as plsc

_N = 10000
_E = 320000
_D = 128
_NC = 2
_NS = 16
_NW = _NC * _NS
_CH = 128
_EPT = _E // _NW
_CPT = 79
_EPAD = _CPT * _CH
_NPAD = 10112
_NSH = 10112
_RPS = _NSH // _NS

_mesh = plsc.VectorSubcoreMesh(core_axis_name="c", subcore_axis_name="s",
                               num_cores=_NC, num_subcores=_NS)

def _s1_body(col_hbm, w_hbm, degp_hbm, col_v, w_v, acc_v, sem):
    cid = lax.axis_index("c")
    sid = lax.axis_index("s")
    wid = sid * _NC + cid
    zero16 = jnp.zeros((16,), jnp.float32)

    def zl(i, _):
        acc_v[pl.ds(i * 16, 16)] = zero16
        return 0
    lax.fori_loop(0, (_NPAD + 16) // 16, zl, 0)

    pltpu.sync_copy(col_hbm.at[wid], col_v)
    pltpu.sync_copy(w_hbm.at[wid], w_v)
    lane = lax.iota(jnp.int32, 16)

    def chunk(j, _):
        def grp(g, _):
            c16 = col_v[j, pl.ds(g * 16, 16)]
            w16 = w_v[j, pl.ds(g * 16, 16)]
            for t in range(16):
                c = c16[t]
                wv = jnp.where(lane == 0, w16[t], jnp.float32(0.0))
                acc_v[pl.ds(c, 16)] = acc_v[pl.ds(c, 16)] + wv
            return 0
        lax.fori_loop(0, _CH // 16, grp, 0)
        return 0
    lax.fori_loop(0, _CPT, chunk, 0)

    obase = pl.multiple_of(wid * _NPAD, 128)
    pltpu.sync_copy(acc_v.at[pl.ds(0, _NPAD)], degp_hbm.at[pl.ds(obase, _NPAD)])


_s1 = pl.kernel(
    _s1_body,
    out_type=jax.ShapeDtypeStruct((_NW * _NPAD,), jnp.float32),
    mesh=_mesh,
    scratch_types=[
        pltpu.VMEM((_CPT, _CH), jnp.int32),
        pltpu.VMEM((_CPT, _CH), jnp.float32),
        pltpu.VMEM((_NPAD + 16,), jnp.float32),
        pltpu.SemaphoreType.DMA,
    ],
)


def _s2_body(y_hbm, row_hbm, col_hbm, w_hbm, out_hbm,
             idxr_v, idxc_v, w_v, rows0_v, acc_sh, gs0):
    cid = lax.axis_index("c")
    sid = lax.axis_index("s")
    wid = sid * _NC + cid
    zero16 = jnp.zeros((16,), jnp.float32)

    def zl(i, _):
        def zl2(q, _):
            rows0_v[i, pl.ds(q * 16, 16)] = zero16
            return 0
        lax.fori_loop(0, _D // 16, zl2, 0)
        return 0
    lax.fori_loop(0, _CH, zl, 0)
    rbase = pl.multiple_of(sid * _RPS, 8)
    for k in range(4):
        pltpu.sync_copy(rows0_v, acc_sh.at[pl.ds(rbase + k * _CH, _CH)])
    pltpu.sync_copy(rows0_v.at[pl.ds(0, _RPS - 4 * _CH)],
                    acc_sh.at[pl.ds(rbase + 4 * _CH, _RPS - 4 * _CH)])
    plsc.subcore_barrier()

    rows_v = rows0_v

    pltpu.sync_copy(row_hbm.at[wid], idxr_v)
    pltpu.sync_copy(col_hbm.at[wid], idxc_v)
    pltpu.sync_copy(w_hbm.at[wid], w_v)

    def chunk(j, _):
        pltpu.async_copy(y_hbm.at[idxr_v.at[j]], rows_v, gs0).wait()

        def grp(g, _):
            w16 = w_v[j, pl.ds(g * 16, 16)]
            for t in range(16):
                e = g * 16 + t
                wt = w16[t]
                for q in range(_D // 16):
                    rows_v[e, pl.ds(q * 16, 16)] = (
                        rows_v[e, pl.ds(q * 16, 16)] * wt)
            return 0
        lax.fori_loop(0, _CH // 16, grp, 0)

        pltpu.sync_copy(rows_v, acc_sh.at[idxc_v.at[j]], add=True)
        return 0
    lax.fori_loop(0, _CPT, chunk, 0)
    plsc.subcore_barrier()

    pltpu.sync_copy(acc_sh.at[pl.ds(rbase, _RPS)],
                    out_hbm.at[cid].at[pl.ds(rbase, _RPS)])


_s2 = pl.kernel(
    _s2_body,
    out_type=jax.ShapeDtypeStruct((_NC, _NSH, _D), jnp.float32),
    mesh=_mesh,
    scratch_types=[
        pltpu.VMEM((_CPT, _CH), jnp.int32),
        pltpu.VMEM((_CPT, _CH), jnp.int32),
        pltpu.VMEM((_CPT, _CH), jnp.float32),
        pltpu.VMEM((_CH, _D), jnp.float32),
        pltpu.VMEM_SHARED((_NSH, _D), jnp.float32),
        pltpu.SemaphoreType.DMA,
    ],
)


def _leaky(v):
    return jnp.where(v >= 0, v, v * jnp.float32(0.01))


def _t1_body(degp_ref, x_ref, y0_ref, dinv_ref):
    ones = jnp.ones((_NW, 1), jnp.float32)
    degc = lax.dot_general(degp_ref[...], ones, (((0,), (0,)), ((), ())),
                           preferred_element_type=jnp.float32)
    deg = degc[:_N]
    dinv = jnp.where(deg > 0, lax.rsqrt(jnp.where(deg > 0, deg, 1.0)), 0.0)
    dinv_ref[...] = dinv
    y0_ref[...] = x_ref[...] * dinv


_t1 = pl.pallas_call(
    _t1_body,
    out_shape=(jax.ShapeDtypeStruct((_N, _D), jnp.float32),
               jax.ShapeDtypeStruct((_N, 1), jnp.float32)),
)


def _t2_body(sp_ref, dinv_ref, h_ref, y_ref):
    dinv = dinv_ref[...]
    h = (sp_ref[0][:_N] + sp_ref[1][:_N]) * dinv
    h_ref[...] = h
    y_ref[...] = h * dinv


_t2 = pl.pallas_call(
    _t2_body,
    out_shape=(jax.ShapeDtypeStruct((_N, _D), jnp.float32),
               jax.ShapeDtypeStruct((_N, _D), jnp.float32)),
)


def _t3_body(x_ref, h1_ref, sp_ref, dinv_ref, w0_ref, w1_ref, w2_ref, b_ref,
             z_ref, yz_ref):
    dinv = dinv_ref[...]
    h2 = (sp_ref[0][:_N] + sp_ref[1][:_N]) * dinv
    acc = jnp.dot(x_ref[...], w0_ref[...], preferred_element_type=jnp.float32)
    acc = acc + jnp.dot(h1_ref[...], w1_ref[...],
                        preferred_element_type=jnp.float32)
    acc = acc + jnp.dot(h2, w2_ref[...], preferred_element_type=jnp.float32)
    z = _leaky(acc + b_ref[...])
    z_ref[...] = z
    yz_ref[...] = z * dinv


_t3 = pl.pallas_call(
    _t3_body,
    out_shape=(jax.ShapeDtypeStruct((_N, _D), jnp.float32),
               jax.ShapeDtypeStruct((_N, _D), jnp.float32)),
)


def _t5_body(z1_ref, g1_ref, sp_ref, dinv_ref, w0_ref, w1_ref, w2_ref, b_ref,
             wc_ref, bc_ref, wp_ref, bp_ref, cl_ref, pw_ref):
    dinv = dinv_ref[...]
    g2 = (sp_ref[0] + sp_ref[1]) * dinv
    acc = jnp.dot(z1_ref[...], w0_ref[...], preferred_element_type=jnp.float32)
    acc = acc + jnp.dot(g1_ref[...], w1_ref[...],
                        preferred_element_type=jnp.float32)
    acc = acc + jnp.dot(g2, w2_ref[...], preferred_element_type=jnp.float32)
    z2 = _leaky(acc + b_ref[...])
    cl_ref[...] = jnp.dot(z2, wc_ref[...],
                          preferred_element_type=jnp.float32) + bc_ref[...]
    pw_ref[...] = jnp.dot(z2, wp_ref[...],
                          preferred_element_type=jnp.float32) + bp_ref[...]


_APS = 1024
_t5 = pl.pallas_call(
    _t5_body,
    out_shape=(jax.ShapeDtypeStruct((_APS, 3), jnp.float32),
               jax.ShapeDtypeStruct((_APS, 3), jnp.float32)),
)


def _pad_seg(a, fill):
    seg = a.reshape(_NW, _EPT)
    return jnp.pad(seg, ((0, 0), (0, _EPAD - _EPT)),
                   constant_values=fill).reshape(_NW, _CPT, _CH)


def kernel(x, edge_index, edge_attr, W1_0, W1_1, W1_2, b1,
           W2_0, W2_1, W2_2, b2, Wc, bc, Wp, bp):
    row = edge_index[0]
    col = edge_index[1]
    w = edge_attr[:, 0]
    perm = jnp.argsort(row)
    row = row[perm]
    col = col[perm]
    w = w[perm]
    rp = _pad_seg(row, 0)
    cp = _pad_seg(col, 0)
    wp = _pad_seg(w, 0.0)

    degp = _s1(cp, wp).reshape(_NW, _NPAD)
    y0, dinv = _t1(degp, x)
    p1 = _s2(y0, rp, cp, wp)
    h1, y1 = _t2(p1, dinv)
    p2 = _s2(y1, rp, cp, wp)
    z1, yz1 = _t3(x, h1, p2, dinv, W1_0, W1_1, W1_2, b1)
    p3 = _s2(yz1, rp, cp, wp)
    g1, yg1 = _t2(p3, dinv)
    p4 = _s2(yg1, rp, cp, wp)
    cl, pw = _t5(z1[:_APS], g1[:_APS], p4[:, :_APS], dinv[:_APS],
                 W2_0, W2_1, W2_2, b2, Wc, bc, Wp, bp)
    return (cl, pw)


# speedup vs baseline: 1.6286x; 1.6286x over previous
import jax
import jax.numpy as jnp
from jax import lax
from jax.experimental import pallas as pl
from jax.experimental.pallas import tpu as pltpu
from jax.experimental.pallas import tpu_sc as plsc

_N = 10000
_E = 320000
_D = 128
_NC = 2
_NS = 16
_NW = _NC * _NS
_CH = 128
_EPT = _E // _NW
_CPT = 79
_EPAD = _CPT * _CH
_NPAD = 10112
_NSH = 10112
_RPS = _NSH // _NS

_mesh = plsc.VectorSubcoreMesh(core_axis_name="c", subcore_axis_name="s",
                               num_cores=_NC, num_subcores=_NS)

def _s1_body(col_hbm, w_hbm, degp_hbm, col_v, w_v, acc_v, sem):
    cid = lax.axis_index("c")
    sid = lax.axis_index("s")
    wid = sid * _NC + cid
    zero16 = jnp.zeros((16,), jnp.float32)

    def zl(i, _):
        acc_v[pl.ds(i * 16, 16)] = zero16
        return 0
    lax.fori_loop(0, (_NPAD + 16) // 16, zl, 0)

    pltpu.sync_copy(col_hbm.at[wid], col_v)
    pltpu.sync_copy(w_hbm.at[wid], w_v)
    lane = lax.iota(jnp.int32, 16)

    def chunk(j, _):
        def grp(g, _):
            c16 = col_v[j, pl.ds(g * 16, 16)]
            w16 = w_v[j, pl.ds(g * 16, 16)]
            for t in range(16):
                c = c16[t]
                wv = jnp.where(lane == 0, w16[t], jnp.float32(0.0))
                acc_v[pl.ds(c, 16)] = acc_v[pl.ds(c, 16)] + wv
            return 0
        lax.fori_loop(0, _CH // 16, grp, 0)
        return 0
    lax.fori_loop(0, _CPT, chunk, 0)

    obase = pl.multiple_of(wid * _NPAD, 128)
    pltpu.sync_copy(acc_v.at[pl.ds(0, _NPAD)], degp_hbm.at[pl.ds(obase, _NPAD)])


_s1 = pl.kernel(
    _s1_body,
    out_type=jax.ShapeDtypeStruct((_NW * _NPAD,), jnp.float32),
    mesh=_mesh,
    scratch_types=[
        pltpu.VMEM((_CPT, _CH), jnp.int32),
        pltpu.VMEM((_CPT, _CH), jnp.float32),
        pltpu.VMEM((_NPAD + 16,), jnp.float32),
        pltpu.SemaphoreType.DMA,
    ],
)


def _s2_body(y_hbm, row_hbm, col_hbm, w_hbm, out_hbm,
             idxr_v, idxc_v, w_v, rows0_v, acc_sh, gs0):
    cid = lax.axis_index("c")
    sid = lax.axis_index("s")
    wid = sid * _NC + cid
    zero16 = jnp.zeros((16,), jnp.float32)

    def zl(i, _):
        def zl2(q, _):
            rows0_v[i, pl.ds(q * 16, 16)] = zero16
            return 0
        lax.fori_loop(0, _D // 16, zl2, 0)
        return 0
    lax.fori_loop(0, _CH, zl, 0)
    rbase = pl.multiple_of(sid * _RPS, 8)
    for k in range(4):
        pltpu.sync_copy(rows0_v, acc_sh.at[pl.ds(rbase + k * _CH, _CH)])
    pltpu.sync_copy(rows0_v.at[pl.ds(0, _RPS - 4 * _CH)],
                    acc_sh.at[pl.ds(rbase + 4 * _CH, _RPS - 4 * _CH)])
    plsc.subcore_barrier()

    rows_v = rows0_v

    pltpu.sync_copy(row_hbm.at[wid], idxr_v)
    pltpu.sync_copy(col_hbm.at[wid], idxc_v)
    pltpu.sync_copy(w_hbm.at[wid], w_v)

    def chunk(j, _):
        pltpu.async_copy(y_hbm.at[idxr_v.at[j]], rows_v, gs0).wait()

        def grp(g, _):
            w16 = w_v[j, pl.ds(g * 16, 16)]
            for t in range(16):
                e = g * 16 + t
                wt = w16[t]
                for q in range(_D // 16):
                    rows_v[e, pl.ds(q * 16, 16)] = (
                        rows_v[e, pl.ds(q * 16, 16)] * wt)
            return 0
        lax.fori_loop(0, _CH // 16, grp, 0)

        pltpu.sync_copy(rows_v, acc_sh.at[idxc_v.at[j]], add=True)
        return 0
    lax.fori_loop(0, _CPT, chunk, 0)
    plsc.subcore_barrier()

    pltpu.sync_copy(acc_sh.at[pl.ds(rbase, _RPS)],
                    out_hbm.at[cid].at[pl.ds(rbase, _RPS)])


_s2 = pl.kernel(
    _s2_body,
    out_type=jax.ShapeDtypeStruct((_NC, _NSH, _D), jnp.float32),
    mesh=_mesh,
    scratch_types=[
        pltpu.VMEM((_CPT, _CH), jnp.int32),
        pltpu.VMEM((_CPT, _CH), jnp.int32),
        pltpu.VMEM((_CPT, _CH), jnp.float32),
        pltpu.VMEM((_CH, _D), jnp.float32),
        pltpu.VMEM_SHARED((_NSH, _D), jnp.float32),
        pltpu.SemaphoreType.DMA,
    ],
)


def _leaky(v):
    return jnp.where(v >= 0, v, v * jnp.float32(0.01))


def _t1_body(degp_ref, x_ref, y0_ref, dinv_ref):
    ones = jnp.ones((_NW, 1), jnp.float32)
    degc = lax.dot_general(degp_ref[...], ones, (((0,), (0,)), ((), ())),
                           preferred_element_type=jnp.float32)
    deg = degc[:_N]
    dinv = jnp.where(deg > 0, lax.rsqrt(jnp.where(deg > 0, deg, 1.0)), 0.0)
    dinv_ref[...] = dinv
    y0_ref[...] = x_ref[...] * dinv


_t1 = pl.pallas_call(
    _t1_body,
    out_shape=(jax.ShapeDtypeStruct((_N, _D), jnp.float32),
               jax.ShapeDtypeStruct((_N, 1), jnp.float32)),
)


def _t2_body(sp_ref, dinv_ref, h_ref, y_ref):
    dinv = dinv_ref[...]
    h = (sp_ref[0][:_N] + sp_ref[1][:_N]) * dinv
    h_ref[...] = h
    y_ref[...] = h * dinv


_t2 = pl.pallas_call(
    _t2_body,
    out_shape=(jax.ShapeDtypeStruct((_N, _D), jnp.float32),
               jax.ShapeDtypeStruct((_N, _D), jnp.float32)),
)


def _t3_body(x_ref, h1_ref, sp_ref, dinv_ref, w0_ref, w1_ref, w2_ref, b_ref,
             z_ref, yz_ref):
    dinv = dinv_ref[...]
    h2 = (sp_ref[0][:_N] + sp_ref[1][:_N]) * dinv
    acc = jnp.dot(x_ref[...], w0_ref[...], preferred_element_type=jnp.float32)
    acc = acc + jnp.dot(h1_ref[...], w1_ref[...],
                        preferred_element_type=jnp.float32)
    acc = acc + jnp.dot(h2, w2_ref[...], preferred_element_type=jnp.float32)
    z = _leaky(acc + b_ref[...])
    z_ref[...] = z
    yz_ref[...] = z * dinv


_t3 = pl.pallas_call(
    _t3_body,
    out_shape=(jax.ShapeDtypeStruct((_N, _D), jnp.float32),
               jax.ShapeDtypeStruct((_N, _D), jnp.float32)),
)


def _t5_body(z1_ref, g1_ref, sp_ref, dinv_ref, w0_ref, w1_ref, w2_ref, b_ref,
             wc_ref, bc_ref, wp_ref, bp_ref, cl_ref, pw_ref):
    dinv = dinv_ref[...]
    g2 = (sp_ref[0] + sp_ref[1]) * dinv
    acc = jnp.dot(z1_ref[...], w0_ref[...], preferred_element_type=jnp.float32)
    acc = acc + jnp.dot(g1_ref[...], w1_ref[...],
                        preferred_element_type=jnp.float32)
    acc = acc + jnp.dot(g2, w2_ref[...], preferred_element_type=jnp.float32)
    z2 = _leaky(acc + b_ref[...])
    cl_ref[...] = jnp.dot(z2, wc_ref[...],
                          preferred_element_type=jnp.float32) + bc_ref[...]
    pw_ref[...] = jnp.dot(z2, wp_ref[...],
                          preferred_element_type=jnp.float32) + bp_ref[...]


_APS = 1024
_t5 = pl.pallas_call(
    _t5_body,
    out_shape=(jax.ShapeDtypeStruct((_APS, 3), jnp.float32),
               jax.ShapeDtypeStruct((_APS, 3), jnp.float32)),
)


def _pad_seg(a, fill):
    seg = a.reshape(_NW, _EPT)
    return jnp.pad(seg, ((0, 0), (0, _EPAD - _EPT)),
                   constant_values=fill).reshape(_NW, _CPT, _CH)


def kernel(x, edge_index, edge_attr, W1_0, W1_1, W1_2, b1,
           W2_0, W2_1, W2_2, b2, Wc, bc, Wp, bp):
    row = edge_index[0]
    col = edge_index[1]
    w = edge_attr[:, 0]
    rp = _pad_seg(row, 0)
    cp = _pad_seg(col, 0)
    wp = _pad_seg(w, 0.0)

    degp = _s1(cp, wp).reshape(_NW, _NPAD)
    y0, dinv = _t1(degp, x)
    p1 = _s2(y0, rp, cp, wp)
    h1, y1 = _t2(p1, dinv)
    p2 = _s2(y1, rp, cp, wp)
    z1, yz1 = _t3(x, h1, p2, dinv, W1_0, W1_1, W1_2, b1)
    p3 = _s2(yz1, rp, cp, wp)
    g1, yg1 = _t2(p3, dinv)
    p4 = _s2(yg1, rp, cp, wp)
    cl, pw = _t5(z1[:_APS], g1[:_APS], p4[:, :_APS], dinv[:_APS],
                 W2_0, W2_1, W2_2, b2, Wc, bc, Wp, bp)
    return (cl, pw)
